# Initial kernel scaffold; baseline (speedup 1.0000x reference)
import jax
import jax.numpy as jnp
from jax.experimental import pallas as pl


def kernel(idx, targets, logits_table):
    raise NotImplementedError("write your pallas kernel here")


# speedup vs baseline: 1.2948x; 1.2948x over previous
import functools

import jax
import jax.numpy as jnp
from jax import lax
from jax.experimental import pallas as pl
from jax.experimental.pallas import tpu as pltpu
from jax.experimental.pallas import tpu_sc as plsc

VOCAB = 1000
BATCH = 1024
SEQ = 50
BL = BATCH * SEQ
NC = 2
NS = 16
NW = NC * NS
ROWS_PER_W = BL // NW
CHUNK = 32
NCHUNKS = ROWS_PER_W // CHUNK
LOGZ_PAD = 1024


def _logz_body(table_ref, out_ref):
    x = table_ref[...]
    m = jnp.max(x, axis=1, keepdims=True)
    s = jnp.sum(jnp.exp(x - m), axis=1, keepdims=True)
    out_ref[...] = m + jnp.log(s)


def _compute_logz(table):
    return pl.pallas_call(
        _logz_body,
        out_shape=jax.ShapeDtypeStruct((VOCAB, 1), jnp.float32),
    )(table)


_mesh = plsc.VectorSubcoreMesh(core_axis_name="c", subcore_axis_name="s")


@functools.partial(
    pl.kernel,
    out_type=[
        jax.ShapeDtypeStruct((BL, VOCAB), jnp.float32),
        jax.ShapeDtypeStruct((NW, 16), jnp.float32),
    ],
    mesh=_mesh,
    compiler_params=pltpu.CompilerParams(
        needs_layout_passes=False, use_tc_tiling_on_sc=False),
    scratch_types=[
        pltpu.VMEM((CHUNK,), jnp.int32),
        pltpu.VMEM((CHUNK,), jnp.int32),
        pltpu.VMEM((CHUNK, VOCAB), jnp.float32),
        pltpu.VMEM((LOGZ_PAD,), jnp.float32),
        pltpu.VMEM((16,), jnp.float32),
        pltpu.SemaphoreType.DMA,
    ],
)
def _sc_main(table_hbm, idx_hbm, tgt_hbm, logz_hbm,
             out_hbm, part_hbm,
             idx_v, tgt_v, rows_v, logz_v, acc_v, sem):
    wid = lax.axis_index("s") * NC + lax.axis_index("c")
    base = wid * ROWS_PER_W
    pltpu.sync_copy(logz_hbm, logz_v)

    def chunk_body(c, acc):
        b0 = base + c * CHUNK
        pltpu.sync_copy(idx_hbm.at[pl.ds(b0, CHUNK)], idx_v)
        pltpu.sync_copy(tgt_hbm.at[pl.ds(b0, CHUNK)], tgt_v)
        pltpu.async_copy(table_hbm.at[idx_v], rows_v, sem).wait()
        pltpu.sync_copy(rows_v, out_hbm.at[pl.ds(b0, CHUNK)])
        for j in range(CHUNK // 16):
            ids = idx_v[pl.ds(j * 16, 16)]
            tgts = tgt_v[pl.ds(j * 16, 16)]
            rowids = lax.iota(jnp.int32, 16) + (j * 16)
            lz = plsc.load_gather(logz_v, [ids])
            pk = plsc.load_gather(rows_v, [rowids, tgts])
            acc = acc + (lz - pk)
        return acc

    acc = lax.fori_loop(0, NCHUNKS, chunk_body, jnp.zeros((16,), jnp.float32))
    acc_v[...] = acc
    pltpu.sync_copy(acc_v, part_hbm.at[wid])


def _loss_body(part_ref, out_ref):
    total = jnp.sum(part_ref[...]) * (1.0 / BL)
    out_ref[...] = jnp.reshape(total, (1, 1))


def _finalize_loss(partials):
    return pl.pallas_call(
        _loss_body,
        out_shape=jax.ShapeDtypeStruct((1, 1), jnp.float32),
    )(partials)


def kernel(idx, targets, logits_table):
    idx_f = idx.reshape(-1).astype(jnp.int32)
    tgt_f = targets.reshape(-1).astype(jnp.int32)
    table = logits_table.astype(jnp.float32)
    logz = _compute_logz(table)
    logz_pad = jnp.pad(logz[:, 0], (0, LOGZ_PAD - VOCAB))
    logits_flat, partials = _sc_main(table, idx_f, tgt_f, logz_pad)
    loss = _finalize_loss(partials)[0, 0]
    return logits_flat.reshape(BATCH, SEQ, VOCAB), loss


# speedup vs baseline: 1.4356x; 1.1088x over previous
import functools

import jax
import jax.numpy as jnp
from jax import lax
from jax.experimental import pallas as pl
from jax.experimental.pallas import tpu as pltpu
from jax.experimental.pallas import tpu_sc as plsc

VOCAB = 1000
BATCH = 1024
SEQ = 50
BL = BATCH * SEQ
NC = 2
NS = 16
NW = NC * NS
ROWS_PER_W = BL // NW
CHUNK = 40
NCHUNKS = ROWS_PER_W // CHUNK
LOGZ_PAD = 1024
IDX_PAD = ROWS_PER_W + 16


def _logz_body(table_ref, out_ref):
    x = table_ref[...]
    m = jnp.max(x, axis=1, keepdims=True)
    s = jnp.sum(jnp.exp(x - m), axis=1, keepdims=True)
    out_ref[...] = m + jnp.log(s)


def _compute_logz(table):
    return pl.pallas_call(
        _logz_body,
        out_shape=jax.ShapeDtypeStruct((VOCAB, 1), jnp.float32),
    )(table)


_mesh = plsc.VectorSubcoreMesh(core_axis_name="c", subcore_axis_name="s")


@functools.partial(
    pl.kernel,
    out_type=[
        jax.ShapeDtypeStruct((BL, VOCAB), jnp.float32),
        jax.ShapeDtypeStruct((NW, 16), jnp.float32),
    ],
    mesh=_mesh,
    compiler_params=pltpu.CompilerParams(
        needs_layout_passes=False, use_tc_tiling_on_sc=False),
    scratch_types=[
        pltpu.VMEM((IDX_PAD,), jnp.int32),
        pltpu.VMEM((IDX_PAD,), jnp.int32),
        pltpu.VMEM((CHUNK, VOCAB), jnp.float32),
        pltpu.VMEM((CHUNK, VOCAB), jnp.float32),
        pltpu.VMEM((LOGZ_PAD,), jnp.float32),
        pltpu.VMEM((16,), jnp.float32),
        pltpu.SemaphoreType.DMA,
        pltpu.SemaphoreType.DMA,
        pltpu.SemaphoreType.DMA,
        pltpu.SemaphoreType.DMA,
    ],
)
def _sc_main(table_hbm, idx_hbm, tgt_hbm, logz_hbm,
             out_hbm, part_hbm,
             idx_v, tgt_v, rows0, rows1, logz_v, acc_v,
             gsem0, gsem1, wsem0, wsem1):
    wid = lax.axis_index("s") * NC + lax.axis_index("c")
    base = wid * ROWS_PER_W
    rows = (rows0, rows1)
    gsems = (gsem0, gsem1)
    wsems = (wsem0, wsem1)

    pltpu.sync_copy(logz_hbm, logz_v)
    pltpu.sync_copy(idx_hbm.at[pl.ds(base, ROWS_PER_W)],
                    idx_v.at[pl.ds(0, ROWS_PER_W)])
    pltpu.sync_copy(tgt_hbm.at[pl.ds(base, ROWS_PER_W)],
                    tgt_v.at[pl.ds(0, ROWS_PER_W)])

    def start_gather(c, b):
        pltpu.async_copy(table_hbm.at[idx_v.at[pl.ds(c * CHUNK, CHUNK)]],
                         rows[b], gsems[b])

    def wait_gather(b):
        pltpu.make_async_copy(table_hbm.at[pl.ds(0, CHUNK)],
                              rows[b], gsems[b]).wait()

    def wait_write(b):
        pltpu.make_async_copy(rows[b], out_hbm.at[pl.ds(0, CHUNK)],
                              wsems[b]).wait()

    def loss_chunk(c, b, acc):
        rbase = c * CHUNK
        for j in range(0, CHUNK, 16):
            nvalid = min(CHUNK - j, 16)
            lanes = lax.iota(jnp.int32, 16)
            ids = idx_v[pl.ds(rbase + j, 16)]
            tgts = tgt_v[pl.ds(rbase + j, 16)]
            rowids = lanes + j
            if nvalid < 16:
                valid = lanes < nvalid
                ids = jnp.where(valid, ids, 0)
                tgts = jnp.where(valid, tgts, 0)
                rowids = jnp.where(valid, rowids, 0)
            lz = plsc.load_gather(logz_v, [ids])
            pk = plsc.load_gather(rows[b], [rowids, tgts])
            nll = lz - pk
            if nvalid < 16:
                nll = jnp.where(valid, nll, 0.0)
            acc = acc + nll
        return acc

    start_gather(0, 0)
    start_gather(1, 1)

    def body(c2, acc):
        for b in range(2):
            c = c2 * 2 + b
            wait_gather(b)
            pltpu.async_copy(rows[b],
                             out_hbm.at[pl.ds(base + c * CHUNK, CHUNK)],
                             wsems[b])
            acc = loss_chunk(c, b, acc)

            @pl.when(c + 2 < NCHUNKS)
            def _():
                wait_write(b)
                start_gather(c + 2, b)

        return acc

    acc = lax.fori_loop(0, NCHUNKS // 2, body,
                        jnp.zeros((16,), jnp.float32))
    wait_write(0)
    wait_write(1)
    acc_v[...] = acc
    pltpu.sync_copy(acc_v, part_hbm.at[wid])


def _loss_body(part_ref, out_ref):
    total = jnp.sum(part_ref[...]) * (1.0 / BL)
    out_ref[...] = jnp.reshape(total, (1, 1))


def _finalize_loss(partials):
    return pl.pallas_call(
        _loss_body,
        out_shape=jax.ShapeDtypeStruct((1, 1), jnp.float32),
    )(partials)


def kernel(idx, targets, logits_table):
    idx_f = idx.reshape(-1).astype(jnp.int32)
    tgt_f = targets.reshape(-1).astype(jnp.int32)
    table = logits_table.astype(jnp.float32)
    logz = _compute_logz(table)
    logz_pad = jnp.pad(logz[:, 0], (0, LOGZ_PAD - VOCAB))
    logits_flat, partials = _sc_main(table, idx_f, tgt_f, logz_pad)
    loss = _finalize_loss(partials)[0, 0]
    return logits_flat.reshape(BATCH, SEQ, VOCAB), loss
